# baseline (speedup 1.0000x reference)
import jax
import jax.numpy as jnp
from jax import lax
from jax.experimental import pallas as pl
from jax.experimental.pallas import tpu as pltpu
from jax.experimental.pallas import tpu_sc as plsc

D_MODEL = 128
N_NODES = 320000
N_GRAPHS = 512

N_TC = 128000
N_SC = N_NODES - N_TC
_TBL = N_GRAPHS * 16


_BN = 16000
_TCH = N_TC // 2
_NB = _TCH // _BN


def _matvec_body(xa_ref, xb_ref, w_ref, b_ref, o_ref):
    dims = (((0,), (1,)), ((), ()))
    sa = lax.dot_general(w_ref[...], xa_ref[...], dims,
                         preferred_element_type=jnp.float32)
    sb = lax.dot_general(w_ref[...], xb_ref[...], dims,
                         preferred_element_type=jnp.float32)
    o_ref[...] = (jnp.concatenate([sa, sb], axis=0) + b_ref[0, 0])[None]


def _node_scalars(x, W, b):
    return pl.pallas_call(
        _matvec_body,
        grid=(_NB,),
        in_specs=[
            pl.BlockSpec((_BN, D_MODEL), lambda i: (i, 0)),
            pl.BlockSpec((_BN, D_MODEL), lambda i: (i + _NB, 0)),
            pl.BlockSpec((D_MODEL, 1), lambda i: (0, 0)),
            pl.BlockSpec((1, 1), lambda i: (0, 0)),
        ],
        out_specs=pl.BlockSpec((1, 2, _BN), lambda i: (i, 0, 0)),
        out_shape=jax.ShapeDtypeStruct((_NB, 2, _BN), jnp.float32),
    )(x, x, W, b.reshape(1, 1)).reshape(N_TC)


_NC = 2
_NT = 16
_NW = _NC * _NT
_CHUNK = N_SC // _NW
_RB = 200
_NCHUNK = _CHUNK // _RB
_SEG_PER_TILE = N_GRAPHS // _NT
_ROW_UNROLL = 5


def _dense_body(x_hbm, ids_hbm, w_hbm, bv_hbm, out_hbm, xb0, xb1, ids_v,
                tbl_v, w_v, bv_v, buf_v, acc_v, out_v, shared,
                sem0, sem1, sem_i):
    cid = lax.axis_index("c")
    sid = lax.axis_index("s")
    wid = cid * _NT + sid
    lane = lax.iota(jnp.int32, 16)
    row0 = N_TC + wid * _CHUNK

    pltpu.async_copy(ids_hbm.at[pl.ds(row0, _CHUNK)], ids_v, sem_i)
    pltpu.sync_copy(w_hbm, w_v)
    pltpu.sync_copy(bv_hbm, bv_v)
    pltpu.async_copy(x_hbm.at[pl.ds(row0, _RB), :], xb0, sem0)
    pltpu.async_copy(x_hbm.at[pl.ds(row0 + _RB, _RB), :], xb1, sem1)

    zeros = jnp.zeros((16,), jnp.float32)

    def _zero(i, c):
        tbl_v[pl.ds(i * 16, 16)] = zeros
        return c

    lax.fori_loop(0, _TBL // 16, _zero, 0)

    ws = [w_v[pl.ds(j * 16, 16)] for j in range(8)]
    bv = bv_v[...]
    pltpu.make_async_copy(ids_hbm.at[pl.ds(row0, _CHUNK)], ids_v,
                          sem_i).wait()

    def _process(chunk, xb):
        @plsc.parallel_loop(0, _RB, unroll=_ROW_UNROLL)
        def _row(rr):
            p = [xb[rr, pl.ds(j * 16, 16)] * ws[j] for j in range(8)]
            q = [p[0] + p[1], p[2] + p[3], p[4] + p[5], p[6] + p[7]]
            acc = ((q[0] + q[1]) + (q[2] + q[3])) + bv
            sid_r = plsc.load_gather(
                ids_v, [lane * 0 + (chunk * _RB + rr)])
            plsc.addupdate_scatter(tbl_v, [sid_r * 16 + lane], acc)

    def _ring(g2, c):
        for bb, (xb, sem) in enumerate(((xb0, sem0), (xb1, sem1))):
            chunk = g2 * 2 + bb
            pltpu.make_async_copy(
                x_hbm.at[pl.ds(row0, _RB), :], xb, sem).wait()
            _process(chunk, xb)

            @pl.when(chunk + 2 < _NCHUNK)
            def _():
                pltpu.async_copy(
                    x_hbm.at[pl.ds(row0 + (chunk + 2) * _RB, _RB), :],
                    xb, sem)
        return c

    lax.fori_loop(0, _NCHUNK // 2, _ring, 0)

    if _NCHUNK % 2:
        pltpu.make_async_copy(
            x_hbm.at[pl.ds(row0, _RB), :], xb0, sem0).wait()
        _process(_NCHUNK - 1, xb0)

    pltpu.sync_copy(tbl_v, shared.at[sid])
    plsc.subcore_barrier()

    def _zacc(i, c):
        acc_v[pl.ds(i * 16, 16)] = zeros
        return c

    lax.fori_loop(0, (_SEG_PER_TILE * 16) // 16, _zacc, 0)

    seg0 = sid * _SEG_PER_TILE
    for src in range(_NT):
        pltpu.sync_copy(
            shared.at[src, pl.ds(seg0 * 16, _SEG_PER_TILE * 16)], buf_v)
        for i in range((_SEG_PER_TILE * 16) // 16):
            sl = pl.ds(i * 16, 16)
            acc_v[sl] = acc_v[sl] + buf_v[sl]

    for v in range(_SEG_PER_TILE // 16):
        r = jnp.zeros((16,), jnp.float32)
        for c in range(16):
            r = r + plsc.load_gather(acc_v, [(lane + v * 16) * 16 + c])
        out_v[pl.ds(v * 16, 16)] = r

    pltpu.sync_copy(out_v, out_hbm.at[cid, pl.ds(seg0, _SEG_PER_TILE)])


def _dense_segsum(x, ids, wv, bv):
    mesh = plsc.VectorSubcoreMesh(
        core_axis_name="c", subcore_axis_name="s", num_cores=_NC)
    return pl.kernel(
        _dense_body,
        out_type=jax.ShapeDtypeStruct((_NC, N_GRAPHS), jnp.float32),
        mesh=mesh,
        compiler_params=pltpu.CompilerParams(needs_layout_passes=False),
        scratch_types=[
            pltpu.VMEM((_RB, D_MODEL), jnp.float32),
            pltpu.VMEM((_RB, D_MODEL), jnp.float32),
            pltpu.VMEM((_CHUNK,), jnp.int32),
            pltpu.VMEM((_TBL,), jnp.float32),
            pltpu.VMEM((D_MODEL,), jnp.float32),
            pltpu.VMEM((16,), jnp.float32),
            pltpu.VMEM((_SEG_PER_TILE * 16,), jnp.float32),
            pltpu.VMEM((_SEG_PER_TILE * 16,), jnp.float32),
            pltpu.VMEM((_SEG_PER_TILE,), jnp.float32),
            pltpu.VMEM_SHARED((_NT, _TBL), jnp.float32),
            pltpu.SemaphoreType.DMA,
            pltpu.SemaphoreType.DMA,
            pltpu.SemaphoreType.DMA,
        ],
    )(x, ids, wv, bv)


_CCH = N_TC // _NT
_NV = _CCH // 16
_C_UNROLL = 5


def _combine_body(s_hbm, ids_hbm, part_hbm, out_hbm, vals_v, ids_v, tbl_v,
                  buf_v, acc_v, out_v, pa_v, pb_v, shared, sem_a, sem_b):
    tid = lax.axis_index("s")
    lane = lax.iota(jnp.int32, 16)
    seg0 = tid * _SEG_PER_TILE

    base = tid * _CCH
    ids_base = (((tid // 2) % 2) * _TCH + (tid // 4) * _BN
                + (tid % 2) * _CCH)
    h_a = pltpu.async_copy(s_hbm.at[pl.ds(base, _CCH)], vals_v, sem_a)
    h_b = pltpu.async_copy(ids_hbm.at[pl.ds(ids_base, _CCH)], ids_v, sem_b)

    zeros = jnp.zeros((16,), jnp.float32)

    def _zero(i, c):
        tbl_v[pl.ds(i * 16, 16)] = zeros
        return c

    lax.fori_loop(0, _TBL // 16, _zero, 0)
    h_a.wait()
    h_b.wait()

    def _scat(i, c):
        for u in range(_C_UNROLL):
            sl = pl.ds((i * _C_UNROLL + u) * 16, 16)
            plsc.addupdate_scatter(tbl_v, [ids_v[sl] * 16 + lane], vals_v[sl])
        return c

    lax.fori_loop(0, _NV // _C_UNROLL, _scat, 0)

    pltpu.sync_copy(tbl_v, shared.at[tid])
    plsc.subcore_barrier()

    def _zacc(i, c):
        acc_v[pl.ds(i * 16, 16)] = zeros
        return c

    lax.fori_loop(0, (_SEG_PER_TILE * 16) // 16, _zacc, 0)

    for src in range(_NT):
        pltpu.sync_copy(
            shared.at[src, pl.ds(seg0 * 16, _SEG_PER_TILE * 16)], buf_v)
        for i in range((_SEG_PER_TILE * 16) // 16):
            sl = pl.ds(i * 16, 16)
            acc_v[sl] = acc_v[sl] + buf_v[sl]

    pltpu.sync_copy(part_hbm.at[0, pl.ds(seg0, _SEG_PER_TILE)], pa_v)
    pltpu.sync_copy(part_hbm.at[1, pl.ds(seg0, _SEG_PER_TILE)], pb_v)

    for v in range(_SEG_PER_TILE // 16):
        r = jnp.zeros((16,), jnp.float32)
        for c in range(16):
            r = r + plsc.load_gather(acc_v, [(lane + v * 16) * 16 + c])
        sl = pl.ds(v * 16, 16)
        out_v[sl] = r + pa_v[sl] + pb_v[sl]

    pltpu.sync_copy(out_v, out_hbm.at[pl.ds(seg0, _SEG_PER_TILE)])


def _combine(s, ids, partials):
    mesh = plsc.VectorSubcoreMesh(
        core_axis_name="c", subcore_axis_name="s", num_cores=1)
    return pl.kernel(
        _combine_body,
        out_type=jax.ShapeDtypeStruct((N_GRAPHS,), jnp.float32),
        mesh=mesh,
        compiler_params=pltpu.CompilerParams(needs_layout_passes=False),
        scratch_types=[
            pltpu.VMEM((_CCH,), jnp.float32),
            pltpu.VMEM((_CCH,), jnp.int32),
            pltpu.VMEM((_TBL,), jnp.float32),
            pltpu.VMEM((_SEG_PER_TILE * 16,), jnp.float32),
            pltpu.VMEM((_SEG_PER_TILE * 16,), jnp.float32),
            pltpu.VMEM((_SEG_PER_TILE,), jnp.float32),
            pltpu.VMEM((_SEG_PER_TILE,), jnp.float32),
            pltpu.VMEM((_SEG_PER_TILE,), jnp.float32),
            pltpu.VMEM_SHARED((_NT, _TBL), jnp.float32),
            pltpu.SemaphoreType.DMA,
            pltpu.SemaphoreType.DMA,
        ],
    )(s, ids, partials)


def kernel(x, batch, W, b):
    ids = batch.astype(jnp.int32)
    wv = W.reshape(D_MODEL)
    bv = jnp.full((16,), b[0] * (1.0 / 16.0), jnp.float32)
    partials = _dense_segsum(x, ids, wv, bv)
    s1 = _node_scalars(x, W, b)
    return _combine(s1, ids, partials)


# speedup vs baseline: 1.0900x; 1.0900x over previous
import jax
import jax.numpy as jnp
from jax import lax
from jax.experimental import pallas as pl
from jax.experimental.pallas import tpu as pltpu
from jax.experimental.pallas import tpu_sc as plsc

D_MODEL = 128
N_NODES = 320000
N_GRAPHS = 512

N_TC = 192000
N_SC = N_NODES - N_TC
_TBL = N_GRAPHS * 16


_BN = 24000
_TCH = N_TC // 2
_NB = _TCH // _BN


def _matvec_body(xa_ref, xb_ref, w_ref, b_ref, o_ref):
    dims = (((0,), (1,)), ((), ()))
    sa = lax.dot_general(w_ref[...], xa_ref[...], dims,
                         preferred_element_type=jnp.float32)
    sb = lax.dot_general(w_ref[...], xb_ref[...], dims,
                         preferred_element_type=jnp.float32)
    o_ref[...] = (jnp.concatenate([sa, sb], axis=0) + b_ref[0, 0])[None]


def _node_scalars(x, W, b):
    return pl.pallas_call(
        _matvec_body,
        grid=(_NB,),
        in_specs=[
            pl.BlockSpec((_BN, D_MODEL), lambda i: (i, 0)),
            pl.BlockSpec((_BN, D_MODEL), lambda i: (i + _NB, 0)),
            pl.BlockSpec((D_MODEL, 1), lambda i: (0, 0)),
            pl.BlockSpec((1, 1), lambda i: (0, 0)),
        ],
        out_specs=pl.BlockSpec((1, 2, _BN), lambda i: (i, 0, 0)),
        out_shape=jax.ShapeDtypeStruct((_NB, 2, _BN), jnp.float32),
    )(x, x, W, b.reshape(1, 1)).reshape(N_TC)


_NC = 2
_NT = 16
_NW = _NC * _NT
_CHUNK = N_SC // _NW
_RB = 200
_NCHUNK = _CHUNK // _RB
_SEG_PER_TILE = N_GRAPHS // _NT
_ROW_UNROLL = 5


def _dense_body(x_hbm, ids_hbm, w_hbm, bv_hbm, out_hbm, xb0, xb1, ids_v,
                tbl_v, w_v, bv_v, buf_v, acc_v, out_v, shared,
                sem0, sem1, sem_i):
    cid = lax.axis_index("c")
    sid = lax.axis_index("s")
    wid = cid * _NT + sid
    lane = lax.iota(jnp.int32, 16)
    row0 = N_TC + wid * _CHUNK

    pltpu.async_copy(ids_hbm.at[pl.ds(row0, _CHUNK)], ids_v, sem_i)
    pltpu.sync_copy(w_hbm, w_v)
    pltpu.sync_copy(bv_hbm, bv_v)
    pltpu.async_copy(x_hbm.at[pl.ds(row0, _RB), :], xb0, sem0)
    pltpu.async_copy(x_hbm.at[pl.ds(row0 + _RB, _RB), :], xb1, sem1)

    zeros = jnp.zeros((16,), jnp.float32)

    def _zero(i, c):
        tbl_v[pl.ds(i * 16, 16)] = zeros
        return c

    lax.fori_loop(0, _TBL // 16, _zero, 0)

    ws = [w_v[pl.ds(j * 16, 16)] for j in range(8)]
    bv = bv_v[...]
    pltpu.make_async_copy(ids_hbm.at[pl.ds(row0, _CHUNK)], ids_v,
                          sem_i).wait()

    def _process(chunk, xb):
        @plsc.parallel_loop(0, _RB, unroll=_ROW_UNROLL)
        def _row(rr):
            p = [xb[rr, pl.ds(j * 16, 16)] * ws[j] for j in range(8)]
            q = [p[0] + p[1], p[2] + p[3], p[4] + p[5], p[6] + p[7]]
            acc = ((q[0] + q[1]) + (q[2] + q[3])) + bv
            sid_r = plsc.load_gather(
                ids_v, [lane * 0 + (chunk * _RB + rr)])
            plsc.addupdate_scatter(tbl_v, [sid_r * 16 + lane], acc)

    def _ring(g2, c):
        for bb, (xb, sem) in enumerate(((xb0, sem0), (xb1, sem1))):
            chunk = g2 * 2 + bb
            pltpu.make_async_copy(
                x_hbm.at[pl.ds(row0, _RB), :], xb, sem).wait()
            _process(chunk, xb)

            @pl.when(chunk + 2 < _NCHUNK)
            def _():
                pltpu.async_copy(
                    x_hbm.at[pl.ds(row0 + (chunk + 2) * _RB, _RB), :],
                    xb, sem)
        return c

    lax.fori_loop(0, _NCHUNK // 2, _ring, 0)

    if _NCHUNK % 2:
        pltpu.make_async_copy(
            x_hbm.at[pl.ds(row0, _RB), :], xb0, sem0).wait()
        _process(_NCHUNK - 1, xb0)

    pltpu.sync_copy(tbl_v, shared.at[sid])
    plsc.subcore_barrier()

    def _zacc(i, c):
        acc_v[pl.ds(i * 16, 16)] = zeros
        return c

    lax.fori_loop(0, (_SEG_PER_TILE * 16) // 16, _zacc, 0)

    seg0 = sid * _SEG_PER_TILE
    for src in range(_NT):
        pltpu.sync_copy(
            shared.at[src, pl.ds(seg0 * 16, _SEG_PER_TILE * 16)], buf_v)
        for i in range((_SEG_PER_TILE * 16) // 16):
            sl = pl.ds(i * 16, 16)
            acc_v[sl] = acc_v[sl] + buf_v[sl]

    for v in range(_SEG_PER_TILE // 16):
        r = jnp.zeros((16,), jnp.float32)
        for c in range(16):
            r = r + plsc.load_gather(acc_v, [(lane + v * 16) * 16 + c])
        out_v[pl.ds(v * 16, 16)] = r

    pltpu.sync_copy(out_v, out_hbm.at[cid, pl.ds(seg0, _SEG_PER_TILE)])


def _dense_segsum(x, ids, wv, bv):
    mesh = plsc.VectorSubcoreMesh(
        core_axis_name="c", subcore_axis_name="s", num_cores=_NC)
    return pl.kernel(
        _dense_body,
        out_type=jax.ShapeDtypeStruct((_NC, N_GRAPHS), jnp.float32),
        mesh=mesh,
        compiler_params=pltpu.CompilerParams(needs_layout_passes=False),
        scratch_types=[
            pltpu.VMEM((_RB, D_MODEL), jnp.float32),
            pltpu.VMEM((_RB, D_MODEL), jnp.float32),
            pltpu.VMEM((_CHUNK,), jnp.int32),
            pltpu.VMEM((_TBL,), jnp.float32),
            pltpu.VMEM((D_MODEL,), jnp.float32),
            pltpu.VMEM((16,), jnp.float32),
            pltpu.VMEM((_SEG_PER_TILE * 16,), jnp.float32),
            pltpu.VMEM((_SEG_PER_TILE * 16,), jnp.float32),
            pltpu.VMEM((_SEG_PER_TILE,), jnp.float32),
            pltpu.VMEM_SHARED((_NT, _TBL), jnp.float32),
            pltpu.SemaphoreType.DMA,
            pltpu.SemaphoreType.DMA,
            pltpu.SemaphoreType.DMA,
        ],
    )(x, ids, wv, bv)


_CCH = N_TC // _NT
_NV = _CCH // 16
_C_UNROLL = 5


def _combine_body(s_hbm, ids_hbm, part_hbm, out_hbm, vals_v, ids_v, tbl_v,
                  buf_v, acc_v, out_v, pa_v, pb_v, shared, sem_a, sem_b):
    tid = lax.axis_index("s")
    lane = lax.iota(jnp.int32, 16)
    seg0 = tid * _SEG_PER_TILE

    base = tid * _CCH
    ids_base = (((tid // 2) % 2) * _TCH + (tid // 4) * _BN
                + (tid % 2) * _CCH)
    h_a = pltpu.async_copy(s_hbm.at[pl.ds(base, _CCH)], vals_v, sem_a)
    h_b = pltpu.async_copy(ids_hbm.at[pl.ds(ids_base, _CCH)], ids_v, sem_b)

    zeros = jnp.zeros((16,), jnp.float32)

    def _zero(i, c):
        tbl_v[pl.ds(i * 16, 16)] = zeros
        return c

    lax.fori_loop(0, _TBL // 16, _zero, 0)
    h_a.wait()
    h_b.wait()

    def _scat(i, c):
        for u in range(_C_UNROLL):
            sl = pl.ds((i * _C_UNROLL + u) * 16, 16)
            plsc.addupdate_scatter(tbl_v, [ids_v[sl] * 16 + lane], vals_v[sl])
        return c

    lax.fori_loop(0, _NV // _C_UNROLL, _scat, 0)

    pltpu.sync_copy(tbl_v, shared.at[tid])
    plsc.subcore_barrier()

    def _zacc(i, c):
        acc_v[pl.ds(i * 16, 16)] = zeros
        return c

    lax.fori_loop(0, (_SEG_PER_TILE * 16) // 16, _zacc, 0)

    for src in range(_NT):
        pltpu.sync_copy(
            shared.at[src, pl.ds(seg0 * 16, _SEG_PER_TILE * 16)], buf_v)
        for i in range((_SEG_PER_TILE * 16) // 16):
            sl = pl.ds(i * 16, 16)
            acc_v[sl] = acc_v[sl] + buf_v[sl]

    pltpu.sync_copy(part_hbm.at[0, pl.ds(seg0, _SEG_PER_TILE)], pa_v)
    pltpu.sync_copy(part_hbm.at[1, pl.ds(seg0, _SEG_PER_TILE)], pb_v)

    for v in range(_SEG_PER_TILE // 16):
        r = jnp.zeros((16,), jnp.float32)
        for c in range(16):
            r = r + plsc.load_gather(acc_v, [(lane + v * 16) * 16 + c])
        sl = pl.ds(v * 16, 16)
        out_v[sl] = r + pa_v[sl] + pb_v[sl]

    pltpu.sync_copy(out_v, out_hbm.at[pl.ds(seg0, _SEG_PER_TILE)])


def _combine(s, ids, partials):
    mesh = plsc.VectorSubcoreMesh(
        core_axis_name="c", subcore_axis_name="s", num_cores=1)
    return pl.kernel(
        _combine_body,
        out_type=jax.ShapeDtypeStruct((N_GRAPHS,), jnp.float32),
        mesh=mesh,
        compiler_params=pltpu.CompilerParams(needs_layout_passes=False),
        scratch_types=[
            pltpu.VMEM((_CCH,), jnp.float32),
            pltpu.VMEM((_CCH,), jnp.int32),
            pltpu.VMEM((_TBL,), jnp.float32),
            pltpu.VMEM((_SEG_PER_TILE * 16,), jnp.float32),
            pltpu.VMEM((_SEG_PER_TILE * 16,), jnp.float32),
            pltpu.VMEM((_SEG_PER_TILE,), jnp.float32),
            pltpu.VMEM((_SEG_PER_TILE,), jnp.float32),
            pltpu.VMEM((_SEG_PER_TILE,), jnp.float32),
            pltpu.VMEM_SHARED((_NT, _TBL), jnp.float32),
            pltpu.SemaphoreType.DMA,
            pltpu.SemaphoreType.DMA,
        ],
    )(s, ids, partials)


def kernel(x, batch, W, b):
    ids = batch.astype(jnp.int32)
    wv = W.reshape(D_MODEL)
    bv = jnp.full((16,), b[0] * (1.0 / 16.0), jnp.float32)
    partials = _dense_segsum(x, ids, wv, bv)
    s1 = _node_scalars(x, W, b)
    return _combine(s1, ids, partials)


# speedup vs baseline: 1.1080x; 1.0165x over previous
import jax
import jax.numpy as jnp
from jax import lax
from jax.experimental import pallas as pl
from jax.experimental.pallas import tpu as pltpu
from jax.experimental.pallas import tpu_sc as plsc

D_MODEL = 128
N_NODES = 320000
N_GRAPHS = 512

N_TC = 224000
N_SC = N_NODES - N_TC
_TBL = N_GRAPHS * 16


_BN = 28000
_TCH = N_TC // 2
_NB = _TCH // _BN


def _matvec_body(xa_ref, xb_ref, w_ref, b_ref, o_ref):
    dims = (((0,), (1,)), ((), ()))
    sa = lax.dot_general(w_ref[...], xa_ref[...], dims,
                         preferred_element_type=jnp.float32)
    sb = lax.dot_general(w_ref[...], xb_ref[...], dims,
                         preferred_element_type=jnp.float32)
    o_ref[...] = (jnp.concatenate([sa, sb], axis=0) + b_ref[0, 0])[None]


def _node_scalars(x, W, b):
    return pl.pallas_call(
        _matvec_body,
        grid=(_NB,),
        in_specs=[
            pl.BlockSpec((_BN, D_MODEL), lambda i: (i, 0)),
            pl.BlockSpec((_BN, D_MODEL), lambda i: (i + _NB, 0)),
            pl.BlockSpec((D_MODEL, 1), lambda i: (0, 0)),
            pl.BlockSpec((1, 1), lambda i: (0, 0)),
        ],
        out_specs=pl.BlockSpec((1, 2, _BN), lambda i: (i, 0, 0)),
        out_shape=jax.ShapeDtypeStruct((_NB, 2, _BN), jnp.float32),
    )(x, x, W, b.reshape(1, 1)).reshape(N_TC)


_NC = 2
_NT = 16
_NW = _NC * _NT
_CHUNK = N_SC // _NW
_RB = 200
_NCHUNK = _CHUNK // _RB
_SEG_PER_TILE = N_GRAPHS // _NT
_ROW_UNROLL = 5


def _dense_body(x_hbm, ids_hbm, w_hbm, bv_hbm, out_hbm, xb0, xb1, ids_v,
                tbl_v, w_v, bv_v, buf_v, acc_v, out_v, shared,
                sem0, sem1, sem_i):
    cid = lax.axis_index("c")
    sid = lax.axis_index("s")
    wid = cid * _NT + sid
    lane = lax.iota(jnp.int32, 16)
    row0 = N_TC + wid * _CHUNK

    pltpu.async_copy(ids_hbm.at[pl.ds(row0, _CHUNK)], ids_v, sem_i)
    pltpu.sync_copy(w_hbm, w_v)
    pltpu.sync_copy(bv_hbm, bv_v)
    pltpu.async_copy(x_hbm.at[pl.ds(row0, _RB), :], xb0, sem0)
    pltpu.async_copy(x_hbm.at[pl.ds(row0 + _RB, _RB), :], xb1, sem1)

    zeros = jnp.zeros((16,), jnp.float32)

    def _zero(i, c):
        tbl_v[pl.ds(i * 16, 16)] = zeros
        return c

    lax.fori_loop(0, _TBL // 16, _zero, 0)

    ws = [w_v[pl.ds(j * 16, 16)] for j in range(8)]
    bv = bv_v[...]
    pltpu.make_async_copy(ids_hbm.at[pl.ds(row0, _CHUNK)], ids_v,
                          sem_i).wait()

    def _process(chunk, xb):
        @plsc.parallel_loop(0, _RB, unroll=_ROW_UNROLL)
        def _row(rr):
            p = [xb[rr, pl.ds(j * 16, 16)] * ws[j] for j in range(8)]
            q = [p[0] + p[1], p[2] + p[3], p[4] + p[5], p[6] + p[7]]
            acc = ((q[0] + q[1]) + (q[2] + q[3])) + bv
            sid_r = plsc.load_gather(
                ids_v, [lane * 0 + (chunk * _RB + rr)])
            plsc.addupdate_scatter(tbl_v, [sid_r * 16 + lane], acc)

    def _ring(g2, c):
        for bb, (xb, sem) in enumerate(((xb0, sem0), (xb1, sem1))):
            chunk = g2 * 2 + bb
            pltpu.make_async_copy(
                x_hbm.at[pl.ds(row0, _RB), :], xb, sem).wait()
            _process(chunk, xb)

            @pl.when(chunk + 2 < _NCHUNK)
            def _():
                pltpu.async_copy(
                    x_hbm.at[pl.ds(row0 + (chunk + 2) * _RB, _RB), :],
                    xb, sem)
        return c

    lax.fori_loop(0, _NCHUNK // 2, _ring, 0)

    if _NCHUNK % 2:
        pltpu.make_async_copy(
            x_hbm.at[pl.ds(row0, _RB), :], xb0, sem0).wait()
        _process(_NCHUNK - 1, xb0)

    pltpu.sync_copy(tbl_v, shared.at[sid])
    plsc.subcore_barrier()

    def _zacc(i, c):
        acc_v[pl.ds(i * 16, 16)] = zeros
        return c

    lax.fori_loop(0, (_SEG_PER_TILE * 16) // 16, _zacc, 0)

    seg0 = sid * _SEG_PER_TILE
    for src in range(_NT):
        pltpu.sync_copy(
            shared.at[src, pl.ds(seg0 * 16, _SEG_PER_TILE * 16)], buf_v)
        for i in range((_SEG_PER_TILE * 16) // 16):
            sl = pl.ds(i * 16, 16)
            acc_v[sl] = acc_v[sl] + buf_v[sl]

    for v in range(_SEG_PER_TILE // 16):
        r = jnp.zeros((16,), jnp.float32)
        for c in range(16):
            r = r + plsc.load_gather(acc_v, [(lane + v * 16) * 16 + c])
        out_v[pl.ds(v * 16, 16)] = r

    pltpu.sync_copy(out_v, out_hbm.at[cid, pl.ds(seg0, _SEG_PER_TILE)])


def _dense_segsum(x, ids, wv, bv):
    mesh = plsc.VectorSubcoreMesh(
        core_axis_name="c", subcore_axis_name="s", num_cores=_NC)
    return pl.kernel(
        _dense_body,
        out_type=jax.ShapeDtypeStruct((_NC, N_GRAPHS), jnp.float32),
        mesh=mesh,
        compiler_params=pltpu.CompilerParams(needs_layout_passes=False),
        scratch_types=[
            pltpu.VMEM((_RB, D_MODEL), jnp.float32),
            pltpu.VMEM((_RB, D_MODEL), jnp.float32),
            pltpu.VMEM((_CHUNK,), jnp.int32),
            pltpu.VMEM((_TBL,), jnp.float32),
            pltpu.VMEM((D_MODEL,), jnp.float32),
            pltpu.VMEM((16,), jnp.float32),
            pltpu.VMEM((_SEG_PER_TILE * 16,), jnp.float32),
            pltpu.VMEM((_SEG_PER_TILE * 16,), jnp.float32),
            pltpu.VMEM((_SEG_PER_TILE,), jnp.float32),
            pltpu.VMEM_SHARED((_NT, _TBL), jnp.float32),
            pltpu.SemaphoreType.DMA,
            pltpu.SemaphoreType.DMA,
            pltpu.SemaphoreType.DMA,
        ],
    )(x, ids, wv, bv)


_CCH = N_TC // _NT
_NV = _CCH // 16
_C_UNROLL = 5


def _combine_body(s_hbm, ids_hbm, part_hbm, out_hbm, vals_v, ids_v, tbl_v,
                  buf_v, acc_v, out_v, pa_v, pb_v, shared, sem_a, sem_b):
    tid = lax.axis_index("s")
    lane = lax.iota(jnp.int32, 16)
    seg0 = tid * _SEG_PER_TILE

    base = tid * _CCH
    ids_base = (((tid // 2) % 2) * _TCH + (tid // 4) * _BN
                + (tid % 2) * _CCH)
    h_a = pltpu.async_copy(s_hbm.at[pl.ds(base, _CCH)], vals_v, sem_a)
    h_b = pltpu.async_copy(ids_hbm.at[pl.ds(ids_base, _CCH)], ids_v, sem_b)

    zeros = jnp.zeros((16,), jnp.float32)

    def _zero(i, c):
        tbl_v[pl.ds(i * 16, 16)] = zeros
        return c

    lax.fori_loop(0, _TBL // 16, _zero, 0)
    h_a.wait()
    h_b.wait()

    def _scat(i, c):
        for u in range(_C_UNROLL):
            sl = pl.ds((i * _C_UNROLL + u) * 16, 16)
            plsc.addupdate_scatter(tbl_v, [ids_v[sl] * 16 + lane], vals_v[sl])
        return c

    lax.fori_loop(0, _NV // _C_UNROLL, _scat, 0)

    pltpu.sync_copy(tbl_v, shared.at[tid])
    plsc.subcore_barrier()

    def _zacc(i, c):
        acc_v[pl.ds(i * 16, 16)] = zeros
        return c

    lax.fori_loop(0, (_SEG_PER_TILE * 16) // 16, _zacc, 0)

    for src in range(_NT):
        pltpu.sync_copy(
            shared.at[src, pl.ds(seg0 * 16, _SEG_PER_TILE * 16)], buf_v)
        for i in range((_SEG_PER_TILE * 16) // 16):
            sl = pl.ds(i * 16, 16)
            acc_v[sl] = acc_v[sl] + buf_v[sl]

    pltpu.sync_copy(part_hbm.at[0, pl.ds(seg0, _SEG_PER_TILE)], pa_v)
    pltpu.sync_copy(part_hbm.at[1, pl.ds(seg0, _SEG_PER_TILE)], pb_v)

    for v in range(_SEG_PER_TILE // 16):
        r = jnp.zeros((16,), jnp.float32)
        for c in range(16):
            r = r + plsc.load_gather(acc_v, [(lane + v * 16) * 16 + c])
        sl = pl.ds(v * 16, 16)
        out_v[sl] = r + pa_v[sl] + pb_v[sl]

    pltpu.sync_copy(out_v, out_hbm.at[pl.ds(seg0, _SEG_PER_TILE)])


def _combine(s, ids, partials):
    mesh = plsc.VectorSubcoreMesh(
        core_axis_name="c", subcore_axis_name="s", num_cores=1)
    return pl.kernel(
        _combine_body,
        out_type=jax.ShapeDtypeStruct((N_GRAPHS,), jnp.float32),
        mesh=mesh,
        compiler_params=pltpu.CompilerParams(needs_layout_passes=False),
        scratch_types=[
            pltpu.VMEM((_CCH,), jnp.float32),
            pltpu.VMEM((_CCH,), jnp.int32),
            pltpu.VMEM((_TBL,), jnp.float32),
            pltpu.VMEM((_SEG_PER_TILE * 16,), jnp.float32),
            pltpu.VMEM((_SEG_PER_TILE * 16,), jnp.float32),
            pltpu.VMEM((_SEG_PER_TILE,), jnp.float32),
            pltpu.VMEM((_SEG_PER_TILE,), jnp.float32),
            pltpu.VMEM((_SEG_PER_TILE,), jnp.float32),
            pltpu.VMEM_SHARED((_NT, _TBL), jnp.float32),
            pltpu.SemaphoreType.DMA,
            pltpu.SemaphoreType.DMA,
        ],
    )(s, ids, partials)


def kernel(x, batch, W, b):
    ids = batch.astype(jnp.int32)
    wv = W.reshape(D_MODEL)
    bv = jnp.full((16,), b[0] * (1.0 / 16.0), jnp.float32)
    partials = _dense_segsum(x, ids, wv, bv)
    s1 = _node_scalars(x, W, b)
    return _combine(s1, ids, partials)
